# baseline (speedup 1.0000x reference)
import functools

import jax
import jax.numpy as jnp
from jax import lax
from jax.experimental import pallas as pl
from jax.experimental.pallas import tpu as pltpu
from jax.experimental.pallas import tpu_sc as plsc

NC = 2
NS = 16
NW = NC * NS
L = 16

N = 200
D = 64
B = 4096
CHUNK = 32 * 8 * 128
NCHUNKS = N * (D // 8)
CPW = NCHUNKS // NW
NBUF = 3
TOT = N * D * B


def _compute(buf, pbv, g):
    splats = [pbv[pl.ds((g * 8 + dl) * L, L)] for dl in range(8)]

    @pl.loop(0, 32, unroll=2)
    def _(bh):
        base = bh * 1024
        for dl in range(8):
            for u8 in range(8):
                plsc.addupdate(
                    buf.at[pl.ds(base + dl * 128 + u8 * L, L)], splats[dl])


def _sc_body(x_hbm, pb_hbm, out_hbm, buf0, buf1, buf2, pb_v,
             si0, si1, si2, so0, so1, so2):
    c = lax.axis_index("c")
    s = lax.axis_index("s")
    w = s * NC + c
    chunk0 = w * CPW
    pltpu.sync_copy(pb_hbm.at[pl.ds(chunk0 * 8 * L, CPW * 8 * L)], pb_v)

    bufs = (buf0, buf1, buf2)
    sin = (si0, si1, si2)
    sout = (so0, so1, so2)

    def in_cp(g, b):
        return pltpu.make_async_copy(
            x_hbm.at[pl.ds((chunk0 + g) * CHUNK, CHUNK)], bufs[b], sin[b])

    def out_cp(g, b):
        return pltpu.make_async_copy(
            bufs[b], out_hbm.at[pl.ds((chunk0 + g) * CHUNK, CHUNK)], sout[b])

    in_cp(0, 0).start()
    in_cp(1, 1).start()

    @pl.loop(0, CPW)
    def _(g):
        for b in range(NBUF):
            @pl.when(g % NBUF == b)
            def _(g=g, b=b):
                @pl.when(g >= 1)
                def _():
                    out_cp(g - 1, (b + NBUF - 1) % NBUF).wait()

                @pl.when(g + 2 < CPW)
                def _():
                    in_cp(g + 2, (b + 2) % NBUF).start()

                in_cp(g, b).wait()
                _compute(bufs[b], pb_v, g)
                out_cp(g, b).start()

    for b in range(NBUF):
        @pl.when((CPW - 1) % NBUF == b)
        def _(b=b):
            out_cp(CPW - 1, b).wait()


_sc_call = functools.partial(
    pl.kernel,
    out_type=jax.ShapeDtypeStruct((TOT,), jnp.float32),
    mesh=plsc.VectorSubcoreMesh(core_axis_name="c", subcore_axis_name="s"),
    scratch_types=[
        pltpu.VMEM((CHUNK,), jnp.float32),
        pltpu.VMEM((CHUNK,), jnp.float32),
        pltpu.VMEM((CHUNK,), jnp.float32),
        pltpu.VMEM((CPW * 8 * L,), jnp.float32),
        pltpu.SemaphoreType.DMA,
        pltpu.SemaphoreType.DMA,
        pltpu.SemaphoreType.DMA,
        pltpu.SemaphoreType.DMA,
        pltpu.SemaphoreType.DMA,
        pltpu.SemaphoreType.DMA,
    ],
)(_sc_body)


def kernel(x, pos_table):
    t = jnp.transpose(x, (1, 2, 0))
    r = t.reshape(N, 8, 8, 32, 128)
    x0 = jnp.transpose(r, (0, 1, 3, 2, 4)).reshape(TOT)

    posf = pos_table[:N].reshape(N * D)
    pb16 = jnp.repeat(posf, L)

    out0 = _sc_call(x0, pb16)

    o = out0.reshape(N, 8, 32, 8, 128)
    o = jnp.transpose(o, (0, 1, 3, 2, 4)).reshape(N, D, B)
    return jnp.transpose(o, (2, 0, 1))
